# baseline (speedup 1.0000x reference)
import functools

import jax
import jax.numpy as jnp
from jax import lax
from jax.experimental import pallas as pl
from jax.experimental.pallas import tpu as pltpu
from jax.experimental.pallas import tpu_sc as plsc

_N = 200
_NPAD = 256
_E = 20000
_NC = 2
_NS = 16
_NW = _NC * _NS
_EB = 624
_J = 6
_GROUPS = _J * 8
_M2 = _NPAD * _NPAD


def _sc_build_dense(src1, dst1, ea1, src2, dst2, ea2):
    mesh = plsc.VectorSubcoreMesh(core_axis_name="c", subcore_axis_name="s")
    chunk = _M2 // _NS

    @functools.partial(
        pl.kernel,
        mesh=mesh,
        out_type=jax.ShapeDtypeStruct((_NC, 2, 2, _M2), jnp.float32),
        scratch_types=[
            pltpu.VMEM_SHARED((_M2,), jnp.float32),
            pltpu.VMEM_SHARED((_M2,), jnp.float32),
            pltpu.VMEM_SHARED((_M2,), jnp.float32),
            pltpu.VMEM_SHARED((_M2,), jnp.float32),
            pltpu.VMEM((768,), jnp.int32),
            pltpu.VMEM((768,), jnp.int32),
            pltpu.VMEM((768,), jnp.float32),
            pltpu.VMEM((_J, 128), jnp.int32),
            pltpu.VMEM((_J, 128), jnp.int32),
            pltpu.VMEM((_J, 128), jnp.float32),
            pltpu.VMEM((chunk,), jnp.float32),
            pltpu.SemaphoreType.DMA,
        ],
    )
    def k(src1_hbm, dst1_hbm, ea1_hbm, src2_hbm, dst2_hbm, ea2_hbm, out_hbm,
          sh_s0, sh_t0, sh_s1, sh_t1,
          src_v, dst_v, val_v, idx_s_v, idx_t_v, val2_v, zeros_v, sem):
        c = lax.axis_index("c")
        s = lax.axis_index("s")
        wid = s * _NC + c
        base = wid * _EB
        nreal = jnp.where(wid == _NW - 1, _EB + 32, _EB)
        shared = (sh_s0, sh_t0, sh_s1, sh_t1)

        def zfill(i, carry):
            zeros_v[pl.ds(i * 16, 16)] = jnp.zeros((16,), jnp.float32)
            return carry

        lax.fori_loop(0, chunk // 16, zfill, 0)
        for r in shared:
            pltpu.async_copy(zeros_v, r.at[pl.ds(s * chunk, chunk)], sem)
        for r in shared:
            pltpu.make_async_copy(zeros_v, r.at[pl.ds(s * chunk, chunk)],
                                  sem).wait()
        plsc.subcore_barrier()

        for g, (sh_, dh_, ea) in enumerate(((src1_hbm, dst1_hbm, ea1_hbm),
                                            (src2_hbm, dst2_hbm, ea2_hbm))):
            pltpu.sync_copy(sh_.at[pl.ds(base, _EB + 32)],
                            src_v.at[pl.ds(0, _EB + 32)])
            pltpu.sync_copy(dh_.at[pl.ds(base, _EB + 32)],
                            dst_v.at[pl.ds(0, _EB + 32)])
            pltpu.sync_copy(ea.at[pl.ds(base, _EB + 32)],
                            val_v.at[pl.ds(0, _EB + 32)])
            for i in range(_GROUPS):
                sl = pl.ds(i * 16, 16)
                row, col = i // 8, pl.ds((i % 8) * 16, 16)
                sv = src_v[sl]
                dv = dst_v[sl]
                vv = val_v[sl]
                if i >= _EB // 16:
                    m = (lax.iota(jnp.int32, 16) + i * 16) < nreal
                    idx_s = jnp.where(m, dv * _NPAD + sv, 0)
                    idx_t = jnp.where(m, sv * _NPAD + dv, 0)
                    vv = jnp.where(m, vv, 0.0)
                else:
                    idx_s = dv * _NPAD + sv
                    idx_t = sv * _NPAD + dv
                idx_s_v[row, col] = idx_s
                idx_t_v[row, col] = idx_t
                val2_v[row, col] = vv
            sh_s = shared[2 * g]
            sh_t = shared[2 * g + 1]
            for j in range(_J):
                pltpu.sync_copy(val2_v.at[j], sh_s.at[idx_s_v.at[j]], add=True)
                pltpu.sync_copy(val2_v.at[j], sh_t.at[idx_t_v.at[j]], add=True)
        plsc.subcore_barrier()

        for g in range(2):
            for m in range(2):
                r = shared[2 * g + m]
                pltpu.async_copy(r.at[pl.ds(s * chunk, chunk)],
                                 out_hbm.at[c, g, m, pl.ds(s * chunk, chunk)],
                                 sem)
        for g in range(2):
            for m in range(2):
                r = shared[2 * g + m]
                pltpu.make_async_copy(
                    r.at[pl.ds(s * chunk, chunk)],
                    out_hbm.at[c, g, m, pl.ds(s * chunk, chunk)], sem).wait()

    return k(src1, dst1, ea1, src2, dst2, ea2)


def _tc_body(s_ref, x1_ref, x2_ref, w1_ref, b1_ref, w2_ref, b2_ref,
             wc1_ref, bc1_ref, wc2_ref, bc2_ref, wc3_ref, bc3_ref,
             o1_ref, o2_ref):
    def mm(a, b):
        return jnp.dot(a, b, preferred_element_type=jnp.float32)

    def mm_hi(a, b):
        return jnp.dot(a, b, preferred_element_type=jnp.float32,
                       precision=lax.Precision.HIGHEST)

    def cheb(a_mat, x, w_ref, bias):
        out = mm(x, w_ref[0])
        tx1 = mm_hi(a_mat, x)
        out = out + mm(tx1, w_ref[1])
        tx2 = 2.0 * mm_hi(a_mat, tx1) - x
        out = out + mm(tx2, w_ref[2])
        return out + bias

    for g, (x_ref, o_ref) in enumerate(((x1_ref, o1_ref), (x2_ref, o2_ref))):
        s_mat = s_ref[0, g, 0] + s_ref[1, g, 0]
        st_mat = s_ref[0, g, 1] + s_ref[1, g, 1]
        deg_row = jnp.sum(s_mat, axis=0, keepdims=True)
        deg_col = jnp.sum(st_mat, axis=1, keepdims=True)
        dinv_row = jnp.where(
            deg_row > 0, 1.0 / jnp.sqrt(jnp.where(deg_row > 0, deg_row, 1.0)),
            0.0)
        dinv_col = jnp.where(
            deg_col > 0, 1.0 / jnp.sqrt(jnp.where(deg_col > 0, deg_col, 1.0)),
            0.0)
        a_mat = -(dinv_col * s_mat * dinv_row)
        a200 = a_mat[:_N, :_N]
        x = x_ref[...]
        h = jnp.maximum(cheb(a200, x, w1_ref, b1_ref[...]), 0.0)
        o = cheb(a200, h, w2_ref, b2_ref[...])
        z = lax.dot_general(o, wc1_ref[...], (((0,), (0,)), ((), ())),
                            preferred_element_type=jnp.float32)
        h1 = jnp.maximum(z + bc1_ref[...], 0.0)
        h2 = jnp.maximum(mm(h1, wc2_ref[...]) + bc2_ref[...], 0.0)
        o_ref[...] = mm(h2, wc3_ref[...]) + bc3_ref[...]


def kernel(x1, edge_index1, edge_attr1, x2, edge_index2, edge_attr2,
           W1, b1, W2, b2, Wc1, bc1, Wc2, bc2, Wc3, bc3):
    ei1 = edge_index1.astype(jnp.int32)
    ei2 = edge_index2.astype(jnp.int32)
    s_flat = _sc_build_dense(ei1[0], ei1[1], edge_attr1,
                             ei2[0], ei2[1], edge_attr2)
    s_all = s_flat.reshape(_NC, 2, 2, _NPAD, _NPAD)
    o1, o2 = pl.pallas_call(
        _tc_body,
        out_shape=[jax.ShapeDtypeStruct((2, 1), jnp.float32),
                   jax.ShapeDtypeStruct((2, 1), jnp.float32)],
    )(s_all, x1, x2, W1, b1, W2, b2, Wc1, bc1, Wc2, bc2, Wc3, bc3)
    return (o1, o2)


# speedup vs baseline: 1.2270x; 1.2270x over previous
import functools

import jax
import jax.numpy as jnp
from jax import lax
from jax.experimental import pallas as pl
from jax.experimental.pallas import tpu as pltpu
from jax.experimental.pallas import tpu_sc as plsc

_N = 200
_NPAD = 256
_E = 20000
_NC = 2
_NS = 16
_NW = _NC * _NS
_EB = 624
_J = 6
_GROUPS = _J * 8
_M2 = _NPAD * _NPAD


def _sc_build_dense(src1, dst1, ea1, src2, dst2, ea2):
    mesh = plsc.VectorSubcoreMesh(core_axis_name="c", subcore_axis_name="s")
    chunk = _M2 // _NS

    @functools.partial(
        pl.kernel,
        mesh=mesh,
        out_type=jax.ShapeDtypeStruct((_NC, 2, _M2), jnp.float32),
        scratch_types=[
            pltpu.VMEM_SHARED((_M2,), jnp.float32),
            pltpu.VMEM_SHARED((_M2,), jnp.float32),
            pltpu.VMEM((768,), jnp.int32),
            pltpu.VMEM((768,), jnp.int32),
            pltpu.VMEM((768,), jnp.float32),
            pltpu.VMEM((768,), jnp.int32),
            pltpu.VMEM((768,), jnp.int32),
            pltpu.VMEM((768,), jnp.float32),
            pltpu.VMEM((_J, 128), jnp.int32),
            pltpu.VMEM((_J, 128), jnp.float32),
            pltpu.VMEM((_J, 128), jnp.int32),
            pltpu.VMEM((_J, 128), jnp.float32),
            pltpu.VMEM((chunk,), jnp.float32),
            pltpu.SemaphoreType.DMA,
            pltpu.SemaphoreType.DMA,
            pltpu.SemaphoreType.DMA,
        ],
    )
    def k(src1_hbm, dst1_hbm, ea1_hbm, src2_hbm, dst2_hbm, ea2_hbm, out_hbm,
          sh_0, sh_1,
          src0_v, dst0_v, val0_v, src1_v, dst1_v, val1_v,
          idx0_v, w0_v, idx1_v, w1_v, zeros_v, sem_in, sem_z, sem_x):
        c = lax.axis_index("c")
        s = lax.axis_index("s")
        wid = s * _NC + c
        base = wid * _EB
        nreal = jnp.where(wid == _NW - 1, _EB + 32, _EB)
        win = pl.ds(base, _EB + 32)
        lo = pl.ds(0, _EB + 32)
        shared = (sh_0, sh_1)
        stage = ((src0_v, dst0_v, val0_v, idx0_v, w0_v),
                 (src1_v, dst1_v, val1_v, idx1_v, w1_v))

        in_cps = []
        for g, (sh_, dh_, ea) in enumerate(((src1_hbm, dst1_hbm, ea1_hbm),
                                            (src2_hbm, dst2_hbm, ea2_hbm))):
            src_v, dst_v, val_v, _, _ = stage[g]
            in_cps.append(pltpu.async_copy(sh_.at[win], src_v.at[lo], sem_in))
            in_cps.append(pltpu.async_copy(dh_.at[win], dst_v.at[lo], sem_in))
            in_cps.append(pltpu.async_copy(ea.at[win], val_v.at[lo], sem_in))

        def zfill(i, carry):
            zeros_v[pl.ds(i * 16, 16)] = jnp.zeros((16,), jnp.float32)
            return carry

        lax.fori_loop(0, chunk // 16, zfill, 0)
        z_cps = [pltpu.async_copy(zeros_v, r.at[pl.ds(s * chunk, chunk)],
                                  sem_z) for r in shared]
        for cp in z_cps:
            cp.wait()
        for cp in in_cps:
            cp.wait()
        plsc.subcore_barrier()

        x_cps = []
        for g in range(2):
            src_v, dst_v, val_v, idx_v, w_v = stage[g]
            for i in range(_GROUPS):
                sl = pl.ds(i * 16, 16)
                row, col = i // 8, pl.ds((i % 8) * 16, 16)
                sv = src_v[sl]
                dv = dst_v[sl]
                vv = val_v[sl]
                if i >= _EB // 16:
                    m = (lax.iota(jnp.int32, 16) + i * 16) < nreal
                    idx = jnp.where(m, dv * _NPAD + sv, 0)
                    vv = jnp.where(m, vv, 0.0)
                else:
                    idx = dv * _NPAD + sv
                idx_v[row, col] = idx
                w_v[row, col] = vv
            for j in range(_J):
                x_cps.append(pltpu.async_copy(
                    w_v.at[j], shared[g].at[idx_v.at[j]], sem_x, add=True))
        for cp in x_cps:
            cp.wait()
        plsc.subcore_barrier()

        o_cps = [pltpu.async_copy(shared[g].at[pl.ds(s * chunk, chunk)],
                                  out_hbm.at[c, g, pl.ds(s * chunk, chunk)],
                                  sem_z)
                 for g in range(2)]
        for cp in o_cps:
            cp.wait()

    return k(src1, dst1, ea1, src2, dst2, ea2)


def _tc_body(s_ref, x1_ref, x2_ref, w1_ref, b1_ref, w2_ref, b2_ref,
             wc1_ref, bc1_ref, wc2_ref, bc2_ref, wc3_ref, bc3_ref,
             o1_ref, o2_ref):
    def mm(a, b):
        return jnp.dot(a, b, preferred_element_type=jnp.float32)

    def mm_t(a, b, precision=None):
        return lax.dot_general(a, b, (((0,), (0,)), ((), ())),
                               preferred_element_type=jnp.float32,
                               precision=precision)

    def mm_hi(a, b):
        return jnp.dot(a, b, preferred_element_type=jnp.float32,
                       precision=lax.Precision.HIGHEST)

    def cheb(a_mat, x, w_ref, bias):
        out = mm(x, w_ref[0])
        tx1 = mm_hi(a_mat, x)
        out = out + mm(tx1, w_ref[1])
        tx2 = 2.0 * mm_hi(a_mat, tx1) - x
        out = out + mm(tx2, w_ref[2])
        return out + bias

    ones11 = jnp.ones((1, 1), jnp.float32)
    for g, (x_ref, o_ref) in enumerate(((x1_ref, o1_ref), (x2_ref, o2_ref))):
        s_mat = s_ref[0, g] + s_ref[1, g]
        deg_row = jnp.sum(s_mat, axis=0, keepdims=True)
        dinv_row = jnp.where(
            deg_row > 0, 1.0 / jnp.sqrt(jnp.where(deg_row > 0, deg_row, 1.0)),
            0.0)
        dinv_col = mm_t(dinv_row, ones11,
                        precision=lax.Precision.HIGHEST)
        a_mat = -(dinv_col * s_mat * dinv_row)
        a200 = a_mat[:_N, :_N]
        x = x_ref[...]
        h = jnp.maximum(cheb(a200, x, w1_ref, b1_ref[...]), 0.0)
        o = cheb(a200, h, w2_ref, b2_ref[...])
        h1 = jnp.maximum(mm_t(o, wc1_ref[...]) + bc1_ref[...], 0.0)
        h2 = jnp.maximum(mm(h1, wc2_ref[...]) + bc2_ref[...], 0.0)
        o_ref[...] = mm(h2, wc3_ref[...]) + bc3_ref[...]


def kernel(x1, edge_index1, edge_attr1, x2, edge_index2, edge_attr2,
           W1, b1, W2, b2, Wc1, bc1, Wc2, bc2, Wc3, bc3):
    ei1 = edge_index1.astype(jnp.int32)
    ei2 = edge_index2.astype(jnp.int32)
    s_flat = _sc_build_dense(ei1[0], ei1[1], edge_attr1,
                             ei2[0], ei2[1], edge_attr2)
    s_all = s_flat.reshape(_NC, 2, _NPAD, _NPAD)
    o1, o2 = pl.pallas_call(
        _tc_body,
        out_shape=[jax.ShapeDtypeStruct((2, 1), jnp.float32),
                   jax.ShapeDtypeStruct((2, 1), jnp.float32)],
    )(s_all, x1, x2, W1, b1, W2, b2, Wc1, bc1, Wc2, bc2, Wc3, bc3)
    return (o1, o2)


# speedup vs baseline: 1.3677x; 1.1146x over previous
import functools

import jax
import jax.numpy as jnp
from jax import lax
from jax.experimental import pallas as pl
from jax.experimental.pallas import tpu as pltpu
from jax.experimental.pallas import tpu_sc as plsc

_N = 200
_NPAD = 256
_E = 20000
_NC = 1
_NS = 16
_NW = _NC * _NS
_EB = _E // _NW // 16 * 16
_XTRA = _E - _EB * _NW
_CAP = -(-(_EB + _XTRA) // 128) * 128
_J = _CAP // 128
_GROUPS = _J * 8
_M2 = _NPAD * _NPAD


def _sc_build_dense(eif1, ea1, eif2, ea2):
    mesh = plsc.VectorSubcoreMesh(core_axis_name="c", subcore_axis_name="s",
                                  num_cores=_NC)
    chunk = _M2 // _NS

    @functools.partial(
        pl.kernel,
        mesh=mesh,
        out_type=jax.ShapeDtypeStruct((_NC, 2, _M2), jnp.float32),
        scratch_types=[
            pltpu.VMEM_SHARED((_M2,), jnp.float32),
            pltpu.VMEM_SHARED((_M2,), jnp.float32),
            pltpu.VMEM((_CAP,), jnp.int32),
            pltpu.VMEM((_CAP,), jnp.int32),
            pltpu.VMEM((_CAP,), jnp.float32),
            pltpu.VMEM((_CAP,), jnp.int32),
            pltpu.VMEM((_CAP,), jnp.int32),
            pltpu.VMEM((_CAP,), jnp.float32),
            pltpu.VMEM((_J, 128), jnp.int32),
            pltpu.VMEM((_J, 128), jnp.float32),
            pltpu.VMEM((_J, 128), jnp.int32),
            pltpu.VMEM((_J, 128), jnp.float32),
            pltpu.VMEM((chunk,), jnp.float32),
            pltpu.SemaphoreType.DMA,
            pltpu.SemaphoreType.DMA,
            pltpu.SemaphoreType.DMA,
        ],
    )
    def k(eif1_hbm, ea1_hbm, eif2_hbm, ea2_hbm, out_hbm,
          sh_0, sh_1,
          src0_v, dst0_v, val0_v, src1_v, dst1_v, val1_v,
          idx0_v, w0_v, idx1_v, w1_v, zeros_v, sem_in, sem_z, sem_x):
        c = lax.axis_index("c")
        s = lax.axis_index("s")
        wid = s * _NC + c
        base = wid * _EB
        nreal = jnp.where(wid == _NW - 1, _EB + _XTRA, _EB)
        shared = (sh_0, sh_1)
        stage = ((src0_v, dst0_v, val0_v, idx0_v, w0_v),
                 (src1_v, dst1_v, val1_v, idx1_v, w1_v))

        in_cps = []
        for g, (eif, ea) in enumerate(((eif1_hbm, ea1_hbm),
                                       (eif2_hbm, ea2_hbm))):
            src_v, dst_v, val_v, _, _ = stage[g]
            in_cps.append(pltpu.async_copy(
                eif.at[pl.ds(base, _CAP)], src_v, sem_in))
            in_cps.append(pltpu.async_copy(
                eif.at[pl.ds(_E + base, _CAP)], dst_v, sem_in))
            in_cps.append(pltpu.async_copy(
                ea.at[pl.ds(base, _CAP)], val_v, sem_in))

        def zfill(i, carry):
            zeros_v[pl.ds(i * 16, 16)] = jnp.zeros((16,), jnp.float32)
            return carry

        lax.fori_loop(0, chunk // 16, zfill, 0)
        z_cps = [pltpu.async_copy(zeros_v, r.at[pl.ds(s * chunk, chunk)],
                                  sem_z) for r in shared]
        for cp in z_cps:
            cp.wait()
        for cp in in_cps:
            cp.wait()
        plsc.subcore_barrier()

        x_cps = []
        for g in range(2):
            src_v, dst_v, val_v, idx_v, w_v = stage[g]
            for i in range(_GROUPS):
                sl = pl.ds(i * 16, 16)
                row, col = i // 8, pl.ds((i % 8) * 16, 16)
                sv = src_v[sl]
                dv = dst_v[sl]
                vv = val_v[sl]
                if i >= _EB // 16:
                    m = (lax.iota(jnp.int32, 16) + i * 16) < nreal
                    idx = jnp.where(m, dv * _NPAD + sv, 0)
                    vv = jnp.where(m, vv, 0.0)
                else:
                    idx = dv * _NPAD + sv
                idx_v[row, col] = idx
                w_v[row, col] = vv
            for j in range(_J):
                x_cps.append(pltpu.async_copy(
                    w_v.at[j], shared[g].at[idx_v.at[j]], sem_x, add=True))
        for cp in x_cps:
            cp.wait()
        plsc.subcore_barrier()

        o_cps = [pltpu.async_copy(shared[g].at[pl.ds(s * chunk, chunk)],
                                  out_hbm.at[c, g, pl.ds(s * chunk, chunk)],
                                  sem_z)
                 for g in range(2)]
        for cp in o_cps:
            cp.wait()

    return k(eif1, ea1, eif2, ea2)


def _tc_body(s_ref, x1_ref, x2_ref, w1_ref, b1_ref, w2_ref, b2_ref,
             wc1_ref, bc1_ref, wc2_ref, bc2_ref, wc3_ref, bc3_ref,
             o1_ref, o2_ref):
    def mm(a, b):
        return jnp.dot(a, b, preferred_element_type=jnp.float32)

    def mm_t(a, b, precision=None):
        return lax.dot_general(a, b, (((0,), (0,)), ((), ())),
                               preferred_element_type=jnp.float32,
                               precision=precision)

    def mm_hi(a, b):
        return jnp.dot(a, b, preferred_element_type=jnp.float32,
                       precision=lax.Precision.HIGHEST)

    def cheb(a_mat, x, w_ref, bias):
        out = mm(x, w_ref[0])
        tx1 = mm_hi(a_mat, x)
        out = out + mm(tx1, w_ref[1])
        tx2 = 2.0 * mm_hi(a_mat, tx1) - x
        out = out + mm(tx2, w_ref[2])
        return out + bias

    ones11 = jnp.ones((1, 1), jnp.float32)
    for g, (x_ref, o_ref) in enumerate(((x1_ref, o1_ref), (x2_ref, o2_ref))):
        s_mat = s_ref[0, g]
        for ci in range(1, _NC):
            s_mat = s_mat + s_ref[ci, g]
        deg_row = jnp.sum(s_mat, axis=0, keepdims=True)
        dinv_row = jnp.where(
            deg_row > 0, 1.0 / jnp.sqrt(jnp.where(deg_row > 0, deg_row, 1.0)),
            0.0)
        dinv_col = mm_t(dinv_row, ones11,
                        precision=lax.Precision.HIGHEST)
        a_mat = -(dinv_col * s_mat * dinv_row)
        a200 = a_mat[:_N, :_N]
        x = x_ref[...]
        h = jnp.maximum(cheb(a200, x, w1_ref, b1_ref[...]), 0.0)
        o = cheb(a200, h, w2_ref, b2_ref[...])
        h1 = jnp.maximum(mm_t(o, wc1_ref[...]) + bc1_ref[...], 0.0)
        h2 = jnp.maximum(mm(h1, wc2_ref[...]) + bc2_ref[...], 0.0)
        o_ref[...] = mm(h2, wc3_ref[...]) + bc3_ref[...]


def kernel(x1, edge_index1, edge_attr1, x2, edge_index2, edge_attr2,
           W1, b1, W2, b2, Wc1, bc1, Wc2, bc2, Wc3, bc3):
    eif1 = edge_index1.astype(jnp.int32).reshape(-1)
    eif2 = edge_index2.astype(jnp.int32).reshape(-1)
    s_flat = _sc_build_dense(eif1, edge_attr1, eif2, edge_attr2)
    s_all = s_flat.reshape(_NC, 2, _NPAD, _NPAD)
    o1, o2 = pl.pallas_call(
        _tc_body,
        out_shape=[jax.ShapeDtypeStruct((2, 1), jnp.float32),
                   jax.ShapeDtypeStruct((2, 1), jnp.float32)],
    )(s_all, x1, x2, W1, b1, W2, b2, Wc1, bc1, Wc2, bc2, Wc3, bc3)
    return (o1, o2)
